# baseline (speedup 1.0000x reference)
import jax
import jax.numpy as jnp
from jax import lax
from jax.experimental import pallas as pl
from jax.experimental.pallas import tpu as pltpu
from jax.experimental.pallas import tpu_sc as plsc

V = 1000000
D = 64
B = 16384
C = 20
NEG = 3
CN = C + NEG
NC = 2
NS = 16
NW = NC * NS

GRS = 400
GRP = GRS // 2
NCHUNK = V // GRS
CSTEPS = -(-NCHUNK // NW)

BPW = B // NW
CHUNK = 16
NSTEPS = BPW // CHUNK


def _repack_body(ctx_hbm, tgt_hbm, cws_hbm, tws_hbm, srca_v, dstp_v, sem):
    cid = lax.axis_index("c")
    sid = lax.axis_index("s")
    wid = sid * NC + cid

    def one_table(tbl_hbm, ws_hbm, ch):
        r0 = pl.multiple_of(ch * GRS, 8)
        k0 = pl.multiple_of(ch * GRP, 8)
        pltpu.sync_copy(tbl_hbm.at[pl.ds(r0, GRS), :], srca_v)

        def row(k, carry):
            for c in range(4):
                dstp_v[k, pl.ds(c * 16, 16)] = srca_v[2 * k, pl.ds(c * 16, 16)]
                dstp_v[k, pl.ds(64 + c * 16, 16)] = \
                    srca_v[2 * k + 1, pl.ds(c * 16, 16)]
            return carry

        lax.fori_loop(0, GRP, row, 0, unroll=False)
        pltpu.sync_copy(dstp_v, ws_hbm.at[pl.ds(k0, GRP), :])

    def step(s, carry):
        ch = s * NW + wid

        @pl.when(ch < NCHUNK)
        def _():
            one_table(ctx_hbm, cws_hbm, ch)
            one_table(tgt_hbm, tws_hbm, ch)

        return carry

    lax.fori_loop(0, CSTEPS, step, 0, unroll=False)


def _gather_body(tidx_hbm, tpar_hbm, cidx_hbm, cpar_hbm,
                 cws_hbm, tws_hbm, out_hbm,
                 tidx_v, tpar_v, cidx_v, cpar_v,
                 tgt_rows_v, ctx_rows_v, out_v, sem_c, sem_t):
    cid = lax.axis_index("c")
    sid = lax.axis_index("s")
    wid = sid * NC + cid
    base = wid * BPW

    def step(s, carry):
        ib = base + s * CHUNK
        pltpu.sync_copy(tidx_hbm.at[pl.ds(ib, CHUNK)], tidx_v)
        pltpu.sync_copy(tpar_hbm.at[pl.ds(ib, CHUNK)],
                        tpar_v.at[pl.ds(0, CHUNK)])
        pltpu.sync_copy(cidx_hbm.at[pl.ds(ib * CN, CHUNK * CN)], cidx_v)
        pltpu.sync_copy(cpar_hbm.at[pl.ds(ib * CN, CHUNK * CN)],
                        cpar_v.at[pl.ds(0, CHUNK * CN)])
        cp_t = pltpu.async_copy(tws_hbm.at[tidx_v], tgt_rows_v, sem_t)
        cp_c = pltpu.async_copy(cws_hbm.at[cidx_v], ctx_rows_v, sem_c)
        cp_t.wait()
        cp_c.wait()

        def item(i, carry2):
            ib23 = i * CN
            pt = tpar_v[pl.ds(i, 16)][0]
            t0 = tgt_rows_v[i, pl.ds(pt, 16)]
            t1 = tgt_rows_v[i, pl.ds(pt + 16, 16)]
            t2 = tgt_rows_v[i, pl.ds(pt + 32, 16)]
            t3 = tgt_rows_v[i, pl.ds(pt + 48, 16)]
            cs0 = jnp.zeros((16,), jnp.float32)
            cs1 = jnp.zeros((16,), jnp.float32)
            cs2 = jnp.zeros((16,), jnp.float32)
            cs3 = jnp.zeros((16,), jnp.float32)
            for j in range(C):
                pc = cpar_v[pl.ds(ib23 + j, 16)][0]
                cs0 = cs0 + ctx_rows_v[ib23 + j, pl.ds(pc, 16)]
                cs1 = cs1 + ctx_rows_v[ib23 + j, pl.ds(pc + 16, 16)]
                cs2 = cs2 + ctx_rows_v[ib23 + j, pl.ds(pc + 32, 16)]
                cs3 = cs3 + ctx_rows_v[ib23 + j, pl.ds(pc + 48, 16)]
            ns0 = jnp.zeros((16,), jnp.float32)
            ns1 = jnp.zeros((16,), jnp.float32)
            ns2 = jnp.zeros((16,), jnp.float32)
            ns3 = jnp.zeros((16,), jnp.float32)
            for j in range(C, CN):
                pn = cpar_v[pl.ds(ib23 + j, 16)][0]
                ns0 = ns0 + ctx_rows_v[ib23 + j, pl.ds(pn, 16)]
                ns1 = ns1 + ctx_rows_v[ib23 + j, pl.ds(pn + 16, 16)]
                ns2 = ns2 + ctx_rows_v[ib23 + j, pl.ds(pn + 32, 16)]
                ns3 = ns3 + ctx_rows_v[ib23 + j, pl.ds(pn + 48, 16)]
            pacc = cs0 * t0 + cs1 * t1 + cs2 * t2 + cs3 * t3
            nacc = ns0 * t0 + ns1 * t1 + ns2 * t2 + ns3 * t3
            out_v[i, pl.ds(0, 16)] = pacc
            out_v[i, pl.ds(16, 16)] = nacc
            return carry2

        lax.fori_loop(0, CHUNK, item, 0, unroll=False)
        pltpu.sync_copy(out_v, out_hbm.at[pl.ds(ib, CHUNK)])
        return carry

    lax.fori_loop(0, NSTEPS, step, 0, unroll=False)


def _tc_body(part_ref, out_ref):
    x = part_ref[...]
    p = jnp.sum(x[:, :16], axis=1) * (1.0 / C)
    n = -jnp.sum(x[:, 16:], axis=1)

    def logsig(v):
        return jnp.minimum(v, 0.0) - jnp.log1p(jnp.exp(-jnp.abs(v)))

    total = jnp.sum(logsig(p) + logsig(n))
    out_ref[0, 0] = -total * (1.0 / B)


def kernel(targets, contexts, negsamples, context_emb, target_emb):
    tidx = targets.astype(jnp.int32)
    cidx = jnp.concatenate(
        [contexts.astype(jnp.int32), negsamples.astype(jnp.int32)],
        axis=1).reshape(B * CN)
    tpacked, tpar = tidx >> 1, (tidx & 1) * 64
    cpacked, cpar = cidx >> 1, (cidx & 1) * 64

    mesh = plsc.VectorSubcoreMesh(core_axis_name="c", subcore_axis_name="s",
                                  num_cores=NC, num_subcores=NS)

    repack = pl.kernel(
        _repack_body,
        out_type=(jax.ShapeDtypeStruct((V // 2, 2 * D), jnp.float32),
                  jax.ShapeDtypeStruct((V // 2, 2 * D), jnp.float32)),
        mesh=mesh,
        scratch_types=[
            pltpu.VMEM((GRS, D), jnp.float32),
            pltpu.VMEM((GRP, 2 * D), jnp.float32),
            pltpu.SemaphoreType.DMA,
        ],
    )
    cws, tws = repack(context_emb, target_emb)

    gather = pl.kernel(
        _gather_body,
        out_type=jax.ShapeDtypeStruct((B, 32), jnp.float32),
        mesh=mesh,
        scratch_types=[
            pltpu.VMEM((CHUNK,), jnp.int32),
            pltpu.VMEM((CHUNK + 16,), jnp.int32),
            pltpu.VMEM((CHUNK * CN,), jnp.int32),
            pltpu.VMEM((CHUNK * CN + 16,), jnp.int32),
            pltpu.VMEM((CHUNK, 2 * D), jnp.float32),
            pltpu.VMEM((CHUNK * CN, 2 * D), jnp.float32),
            pltpu.VMEM((CHUNK, 32), jnp.float32),
            pltpu.SemaphoreType.DMA,
            pltpu.SemaphoreType.DMA,
        ],
    )
    part = gather(tpacked, tpar, cpacked, cpar, cws, tws)

    loss = pl.pallas_call(
        _tc_body,
        out_shape=jax.ShapeDtypeStruct((1, 1), jnp.float32),
        in_specs=[pl.BlockSpec(memory_space=pltpu.VMEM)],
        out_specs=pl.BlockSpec(memory_space=pltpu.SMEM),
    )(part)
    return loss


# speedup vs baseline: 2.2258x; 2.2258x over previous
import jax
import jax.numpy as jnp
from jax import lax
from jax.experimental import pallas as pl
from jax.experimental.pallas import tpu as pltpu
from jax.experimental.pallas import tpu_sc as plsc

V = 1000000
D = 64
B = 16384
C = 20
NEG = 3
CN = C + NEG
NC = 2
NS = 16
NW = NC * NS
GRS = 160
GRP = GRS // 2
NCHUNK = V // GRS
CSTEPS = -(-NCHUNK // NW)
BPW = B // NW
CHUNK = 16
NSTEPS = BPW // CHUNK


def _repack_body(ctx_hbm, cws_hbm,
                 srca_v, srcb_v, dsta_v, dstb_v,
                 semi_a, semi_b, semo_a, semo_b):
    cid = lax.axis_index("c")
    sid = lax.axis_index("s")
    wid = sid * NC + cid

    srcs = (srca_v, srcb_v)
    dsts = (dsta_v, dstb_v)
    semis = (semi_a, semi_b)
    semos = (semo_a, semo_b)

    def chunk_of(s):
        return s * NW + wid

    def issue_in(s, b):
        ch = chunk_of(s)

        @pl.when(ch < NCHUNK)
        def _():
            r0 = pl.multiple_of(ch * GRS, 8)
            pltpu.async_copy(ctx_hbm.at[pl.ds(r0, GRS), :], srcs[b], semis[b])

    def work(s, b):
        ch = chunk_of(s)

        @pl.when(jnp.logical_and(s >= 2, chunk_of(s - 2) < NCHUNK))
        def _():
            pltpu.make_async_copy(dsts[b], cws_hbm.at[pl.ds(0, GRP), :],
                                  semos[b]).wait()

        @pl.when(ch < NCHUNK)
        def _():
            pltpu.make_async_copy(ctx_hbm.at[pl.ds(0, GRS), :],
                                  srcs[b], semis[b]).wait()
            sv = srcs[b]
            dv = dsts[b]

            def row(k, carry):
                for c in range(4):
                    dv[k, pl.ds(c * 16, 16)] = sv[2 * k, pl.ds(c * 16, 16)]
                    dv[k, pl.ds(64 + c * 16, 16)] = \
                        sv[2 * k + 1, pl.ds(c * 16, 16)]
                return carry

            lax.fori_loop(0, GRP, row, 0, unroll=4)
            k0 = pl.multiple_of(ch * GRP, 8)
            pltpu.async_copy(dv, cws_hbm.at[pl.ds(k0, GRP), :], semos[b])

    issue_in(0, 0)

    def step2(s2, carry):
        s = s2 * 2
        issue_in(s + 1, 1)
        work(s, 0)
        issue_in(s + 2, 0)
        work(s + 1, 1)
        return carry

    lax.fori_loop(0, (CSTEPS + 1) // 2, step2, 0, unroll=False)

    def tail(s, b):
        @pl.when(chunk_of(s) < NCHUNK)
        def _():
            pltpu.make_async_copy(dsts[b], cws_hbm.at[pl.ds(0, GRP), :],
                                  semos[b]).wait()

    ep = 2 * ((CSTEPS + 1) // 2)
    tail(ep - 2, 0)
    tail(ep - 1, 1)


def _gather_body(tidx_hbm, tsub_hbm, cidx_hbm, cpar_hbm, cws_hbm,
                 tgt_emb_hbm, out_hbm,
                 tidx_v, tsub_v, cidx_v, cpar_v,
                 tgt_rows_v, ctx_rows_v, out_v, sem_c, sem_t):
    cid = lax.axis_index("c")
    sid = lax.axis_index("s")
    wid = sid * NC + cid
    base = wid * BPW

    pltpu.sync_copy(tidx_hbm.at[pl.ds(base, BPW)],
                    tidx_v.at[pl.ds(0, BPW)])
    pltpu.sync_copy(tsub_hbm.at[pl.ds(base, BPW)],
                    tsub_v.at[pl.ds(0, BPW)])
    pltpu.sync_copy(cidx_hbm.at[pl.ds(base * CN, BPW * CN)], cidx_v)
    pltpu.sync_copy(cpar_hbm.at[pl.ds(base * CN, BPW * CN)],
                    cpar_v.at[pl.ds(0, BPW * CN)])

    def step(s, carry):
        ib = base + s * CHUNK
        cp_c = pltpu.async_copy(
            cws_hbm.at[cidx_v.at[pl.ds(s * (CHUNK * CN), CHUNK * CN)]],
            ctx_rows_v, sem_c)

        def issue_tgt(i, carry2):
            g = tidx_v[pl.ds(s * CHUNK + i, 16)][0]
            r0 = pl.multiple_of(g * 8, 8)
            pltpu.async_copy(tgt_emb_hbm.at[pl.ds(r0, 8), :],
                             tgt_rows_v.at[i], sem_t)
            return carry2

        lax.fori_loop(0, CHUNK, issue_tgt, 0, unroll=False)
        pltpu.make_async_copy(tgt_emb_hbm.at[pl.ds(0, CHUNK * 8), :],
                              tgt_rows_v, sem_t).wait()
        cp_c.wait()

        def item(i, carry2):
            ib23 = i * CN
            gi23 = s * (CHUNK * CN) + ib23
            su = tsub_v[pl.ds(s * CHUNK + i, 16)][0]
            t0 = tgt_rows_v[i, su, pl.ds(0, 16)]
            t1 = tgt_rows_v[i, su, pl.ds(16, 16)]
            t2 = tgt_rows_v[i, su, pl.ds(32, 16)]
            t3 = tgt_rows_v[i, su, pl.ds(48, 16)]
            cs0 = jnp.zeros((16,), jnp.float32)
            cs1 = jnp.zeros((16,), jnp.float32)
            cs2 = jnp.zeros((16,), jnp.float32)
            cs3 = jnp.zeros((16,), jnp.float32)
            for j in range(C):
                pc = cpar_v[pl.ds(gi23 + j, 16)][0]
                cs0 = cs0 + ctx_rows_v[ib23 + j, pl.ds(pc, 16)]
                cs1 = cs1 + ctx_rows_v[ib23 + j, pl.ds(pc + 16, 16)]
                cs2 = cs2 + ctx_rows_v[ib23 + j, pl.ds(pc + 32, 16)]
                cs3 = cs3 + ctx_rows_v[ib23 + j, pl.ds(pc + 48, 16)]
            ns0 = jnp.zeros((16,), jnp.float32)
            ns1 = jnp.zeros((16,), jnp.float32)
            ns2 = jnp.zeros((16,), jnp.float32)
            ns3 = jnp.zeros((16,), jnp.float32)
            for j in range(C, CN):
                pn = cpar_v[pl.ds(gi23 + j, 16)][0]
                ns0 = ns0 + ctx_rows_v[ib23 + j, pl.ds(pn, 16)]
                ns1 = ns1 + ctx_rows_v[ib23 + j, pl.ds(pn + 16, 16)]
                ns2 = ns2 + ctx_rows_v[ib23 + j, pl.ds(pn + 32, 16)]
                ns3 = ns3 + ctx_rows_v[ib23 + j, pl.ds(pn + 48, 16)]
            pacc = cs0 * t0 + cs1 * t1 + cs2 * t2 + cs3 * t3
            nacc = ns0 * t0 + ns1 * t1 + ns2 * t2 + ns3 * t3
            out_v[i, pl.ds(0, 16)] = pacc
            out_v[i, pl.ds(16, 16)] = nacc
            return carry2

        lax.fori_loop(0, CHUNK, item, 0, unroll=False)
        pltpu.sync_copy(out_v, out_hbm.at[pl.ds(ib, CHUNK)])
        return carry

    lax.fori_loop(0, NSTEPS, step, 0, unroll=False)


def _tc_body(part_ref, out_ref):
    x = part_ref[...]
    p = jnp.sum(x[:, :16], axis=1) * (1.0 / C)
    n = -jnp.sum(x[:, 16:], axis=1)

    def logsig(v):
        return jnp.minimum(v, 0.0) - jnp.log1p(jnp.exp(-jnp.abs(v)))

    total = jnp.sum(logsig(p) + logsig(n))
    out_ref[0, 0] = -total * (1.0 / B)


def kernel(targets, contexts, negsamples, context_emb, target_emb):
    tidx = targets.astype(jnp.int32)
    tgrp, tsub = tidx >> 3, tidx & 7
    cidx = jnp.concatenate(
        [contexts.astype(jnp.int32), negsamples.astype(jnp.int32)],
        axis=1).reshape(B * CN)
    cpacked, cpar = cidx >> 1, (cidx & 1) * 64

    mesh = plsc.VectorSubcoreMesh(core_axis_name="c", subcore_axis_name="s",
                                  num_cores=NC, num_subcores=NS)

    repack = pl.kernel(
        _repack_body,
        out_type=jax.ShapeDtypeStruct((V // 2, 2 * D), jnp.float32),
        mesh=mesh,
        scratch_types=[
            pltpu.VMEM((GRS, D), jnp.float32),
            pltpu.VMEM((GRS, D), jnp.float32),
            pltpu.VMEM((GRP, 2 * D), jnp.float32),
            pltpu.VMEM((GRP, 2 * D), jnp.float32),
            pltpu.SemaphoreType.DMA,
            pltpu.SemaphoreType.DMA,
            pltpu.SemaphoreType.DMA,
            pltpu.SemaphoreType.DMA,
        ],
    )
    cws = repack(context_emb)

    gather = pl.kernel(
        _gather_body,
        out_type=jax.ShapeDtypeStruct((B, 32), jnp.float32),
        mesh=mesh,
        scratch_types=[
            pltpu.VMEM((BPW + 16,), jnp.int32),
            pltpu.VMEM((BPW + 16,), jnp.int32),
            pltpu.VMEM((BPW * CN,), jnp.int32),
            pltpu.VMEM((BPW * CN + 16,), jnp.int32),
            pltpu.VMEM((CHUNK, 8, D), jnp.float32),
            pltpu.VMEM((CHUNK * CN, 2 * D), jnp.float32),
            pltpu.VMEM((CHUNK, 32), jnp.float32),
            pltpu.SemaphoreType.DMA,
            pltpu.SemaphoreType.DMA,
        ],
    )
    part = gather(tgrp, tsub, cpacked, cpar, cws, target_emb)

    loss = pl.pallas_call(
        _tc_body,
        out_shape=jax.ShapeDtypeStruct((1, 1), jnp.float32),
        in_specs=[pl.BlockSpec(memory_space=pltpu.VMEM)],
        out_specs=pl.BlockSpec(memory_space=pltpu.SMEM),
    )(part)
    return loss


# speedup vs baseline: 2.4331x; 1.0931x over previous
import jax
import jax.numpy as jnp
from jax import lax
from jax.experimental import pallas as pl
from jax.experimental.pallas import tpu as pltpu
from jax.experimental.pallas import tpu_sc as plsc

V = 1000000
D = 64
B = 16384
C = 20
NEG = 3
CN = C + NEG
NC = 2
NS = 16
NW = NC * NS
BPW = B // NW
CHUNK = 8
NSTEPS = BPW // CHUNK


def _gather_body(tidx_hbm, cidx_hbm, ctx_tbl_hbm, tgt_emb_hbm, out_hbm,
                 tidx_v, cidx_v, cpk0_v, cpk1_v,
                 tgt_rows0, tgt_rows1, ctx_rows0, ctx_rows1,
                 out_v0, out_v1, sem_c0, sem_c1, sem_t0, sem_t1):
    cid = lax.axis_index("c")
    sid = lax.axis_index("s")
    wid = sid * NC + cid
    base = wid * BPW

    pltpu.sync_copy(tidx_hbm.at[pl.ds(base, BPW)],
                    tidx_v.at[pl.ds(0, BPW)])
    pltpu.sync_copy(cidx_hbm.at[pl.ds(base * CN, BPW * CN)],
                    cidx_v.at[pl.ds(0, BPW * CN)])

    tgt_rows = (tgt_rows0, tgt_rows1)
    ctx_rows = (ctx_rows0, ctx_rows1)
    outs = (out_v0, out_v1)
    sems_c = (sem_c0, sem_c1)
    sems_t = (sem_t0, sem_t1)
    cpks = (cpk0_v, cpk1_v)

    def issue(s, b):
        s0 = s * (CHUNK * CN)
        for w in range(CHUNK * CN // 16):
            cpks[b][pl.ds(w * 16, 16)] = \
                lax.shift_right_logical(cidx_v[pl.ds(s0 + w * 16, 16)], 1)
        if CHUNK * CN % 16:
            tl = CHUNK * CN - 16
            cpks[b][pl.ds(tl, 16)] = \
                lax.shift_right_logical(cidx_v[pl.ds(s0 + tl, 16)], 1)
        pltpu.async_copy(ctx_tbl_hbm.at[cpks[b]], ctx_rows[b], sems_c[b])

        def issue_tgt(i, carry2):
            t = tidx_v[pl.ds(s * CHUNK + i, 16)][0]
            r0 = pl.multiple_of((t >> 3) * 8, 8)
            pltpu.async_copy(tgt_emb_hbm.at[pl.ds(r0, 8), :],
                             tgt_rows[b].at[i], sems_t[b])
            return carry2

        lax.fori_loop(0, CHUNK, issue_tgt, 0, unroll=False)

    def wait_bufs(b):
        pltpu.make_async_copy(tgt_emb_hbm.at[pl.ds(0, CHUNK * 8), :],
                              tgt_rows[b], sems_t[b]).wait()
        pltpu.make_async_copy(ctx_tbl_hbm.at[cpks[b]],
                              ctx_rows[b], sems_c[b]).wait()

    def step_b(s, b):
        ib = base + s * CHUNK

        @pl.when(s + 1 < NSTEPS)
        def _():
            issue(s + 1, 1 - b)

        wait_bufs(b)
        trv = tgt_rows[b]
        crv = ctx_rows[b]
        ov = outs[b]

        def item(i, carry2):
            ib23 = i * CN
            gi23 = s * (CHUNK * CN) + ib23
            t = tidx_v[pl.ds(s * CHUNK + i, 16)][0]
            su = t & 7
            t0 = trv[i, su, pl.ds(0, 16)]
            t1 = trv[i, su, pl.ds(16, 16)]
            t2 = trv[i, su, pl.ds(32, 16)]
            t3 = trv[i, su, pl.ds(48, 16)]
            cs0 = jnp.zeros((16,), jnp.float32)
            cs1 = jnp.zeros((16,), jnp.float32)
            cs2 = jnp.zeros((16,), jnp.float32)
            cs3 = jnp.zeros((16,), jnp.float32)
            for j in range(C):
                pc = (cidx_v[pl.ds(gi23 + j, 16)][0] & 1) * 64
                cs0 = cs0 + crv[ib23 + j, pl.ds(pc, 16)]
                cs1 = cs1 + crv[ib23 + j, pl.ds(pc + 16, 16)]
                cs2 = cs2 + crv[ib23 + j, pl.ds(pc + 32, 16)]
                cs3 = cs3 + crv[ib23 + j, pl.ds(pc + 48, 16)]
            ns0 = jnp.zeros((16,), jnp.float32)
            ns1 = jnp.zeros((16,), jnp.float32)
            ns2 = jnp.zeros((16,), jnp.float32)
            ns3 = jnp.zeros((16,), jnp.float32)
            for j in range(C, CN):
                pn = (cidx_v[pl.ds(gi23 + j, 16)][0] & 1) * 64
                ns0 = ns0 + crv[ib23 + j, pl.ds(pn, 16)]
                ns1 = ns1 + crv[ib23 + j, pl.ds(pn + 16, 16)]
                ns2 = ns2 + crv[ib23 + j, pl.ds(pn + 32, 16)]
                ns3 = ns3 + crv[ib23 + j, pl.ds(pn + 48, 16)]
            pacc = cs0 * t0 + cs1 * t1 + cs2 * t2 + cs3 * t3
            nacc = ns0 * t0 + ns1 * t1 + ns2 * t2 + ns3 * t3
            ov[i, pl.ds(0, 16)] = pacc
            ov[i, pl.ds(16, 16)] = nacc
            return carry2

        lax.fori_loop(0, CHUNK, item, 0, unroll=False)
        pltpu.sync_copy(ov, out_hbm.at[pl.ds(ib, CHUNK)])

    issue(0, 0)

    def step(s, carry):
        b = lax.rem(s, 2)

        @pl.when(b == 0)
        def _():
            step_b(s, 0)

        @pl.when(b == 1)
        def _():
            step_b(s, 1)

        return carry

    lax.fori_loop(0, NSTEPS, step, 0, unroll=False)


def _tc_body(part_ref, out_ref):
    x = part_ref[...]
    p = jnp.sum(x[:, :16], axis=1) * (1.0 / C)
    n = -jnp.sum(x[:, 16:], axis=1)

    def logsig(v):
        return jnp.minimum(v, 0.0) - jnp.log1p(jnp.exp(-jnp.abs(v)))

    total = jnp.sum(logsig(p) + logsig(n))
    out_ref[0, 0] = -total * (1.0 / B)


def kernel(targets, contexts, negsamples, context_emb, target_emb):
    tidx = targets.astype(jnp.int32)
    cidx = jnp.concatenate(
        [contexts.astype(jnp.int32), negsamples.astype(jnp.int32)],
        axis=1).reshape(B * CN)
    ctx_tbl = context_emb.reshape(V // 2, 2 * D)

    mesh = plsc.VectorSubcoreMesh(core_axis_name="c", subcore_axis_name="s",
                                  num_cores=NC, num_subcores=NS)
    gather = pl.kernel(
        _gather_body,
        out_type=jax.ShapeDtypeStruct((B, 32), jnp.float32),
        mesh=mesh,
        scratch_types=[
            pltpu.VMEM((BPW + 16,), jnp.int32),
            pltpu.VMEM((BPW * CN + 16,), jnp.int32),
            pltpu.VMEM((CHUNK * CN,), jnp.int32),
            pltpu.VMEM((CHUNK * CN,), jnp.int32),
            pltpu.VMEM((CHUNK, 8, D), jnp.float32),
            pltpu.VMEM((CHUNK, 8, D), jnp.float32),
            pltpu.VMEM((CHUNK * CN, 2 * D), jnp.float32),
            pltpu.VMEM((CHUNK * CN, 2 * D), jnp.float32),
            pltpu.VMEM((CHUNK, 32), jnp.float32),
            pltpu.VMEM((CHUNK, 32), jnp.float32),
            pltpu.SemaphoreType.DMA,
            pltpu.SemaphoreType.DMA,
            pltpu.SemaphoreType.DMA,
            pltpu.SemaphoreType.DMA,
        ],
    )
    part = gather(tidx, cidx, ctx_tbl, target_emb)

    loss = pl.pallas_call(
        _tc_body,
        out_shape=jax.ShapeDtypeStruct((1, 1), jnp.float32),
        in_specs=[pl.BlockSpec(memory_space=pltpu.VMEM)],
        out_specs=pl.BlockSpec(memory_space=pltpu.SMEM),
    )(part)
    return loss
